# baseline (speedup 1.0000x reference)
import functools

import jax
import jax.numpy as jnp
from jax import lax
from jax.experimental import pallas as pl
from jax.experimental.pallas import tpu as pltpu
from jax.experimental.pallas import tpu_sc as plsc

N = 4096
D = 128
CUT = 0.125
NB = 16
OCC = 163840

N_TILES = 32
CHUNK = 128
BAND = N // N_TILES
SLAB = 6272
ECAP = N_TILES * SLAB
NWORK = 6144
C2EPS = CUT * CUT + 1e-4


def _prefilter_body(rblk_ref, rt_ref, p_ref, o_ref):
    rblk = rblk_ref[...]
    rt = rt_ref[...]
    r2row = jnp.sum(rblk * rblk, axis=1, keepdims=True)
    r2col = jnp.sum(rt * rt, axis=0, keepdims=True)
    dot = jnp.dot(rblk, rt, preferred_element_type=jnp.float32)
    dist2 = (r2row + r2col) - 2.0 * dot
    m = (dist2 < C2EPS).astype(jnp.float32)
    cnt = jnp.dot(m, p_ref[...], preferred_element_type=jnp.float32)
    o_ref[...] = cnt.astype(jnp.int32)


def _prefilter(r, rt, p):
    rows = 256
    return pl.pallas_call(
        _prefilter_body,
        grid=(N // rows,),
        in_specs=[
            pl.BlockSpec((rows, 3), lambda i: (i, 0)),
            pl.BlockSpec((3, N), lambda i: (0, 0)),
            pl.BlockSpec((N, N // 16), lambda i: (0, 0)),
        ],
        out_specs=pl.BlockSpec((rows, N // 16), lambda i: (i, 0)),
        out_shape=jax.ShapeDtypeStruct((N, N // 16), jnp.int32),
    )(r, rt, p)


def _edges_body(xs_ref, ys_ref, zs_ref, cnts_ref, snd_ref, rcv_ref, d2_ref,
                ecnt_ref, x_v, y_v, z_v, r2_v, xb_v, yb_v, zb_v, cl_v, wl_v,
                sndl_v, rcvl_v, d2l_v, ew_v, sem):
    cid = lax.axis_index("c")
    sid = lax.axis_index("s")
    tile = cid * 16 + sid
    r0 = tile * BAND

    pltpu.sync_copy(xs_ref, x_v.at[pl.ds(0, N)])
    pltpu.sync_copy(ys_ref, y_v.at[pl.ds(0, N)])
    pltpu.sync_copy(zs_ref, z_v.at[pl.ds(0, N)])
    pltpu.sync_copy(cnts_ref.at[pl.ds(r0, BAND)], cl_v)

    zeros16 = jnp.zeros((16,), jnp.int32)
    iota16 = lax.iota(jnp.int32, 16)

    def bf16r(v):
        b = plsc.bitcast(v, jnp.int32)
        b = (b + 0x7FFF + ((b >> 16) & 1)) & ~0xFFFF
        return plsc.bitcast(b, jnp.float32)

    def r2row(q, _):
        s = pl.ds(16 * q, 16)
        x = x_v[s]
        y = y_v[s]
        z = z_v[s]
        r2_v[s] = x * x + y * y + z * z
        xb_v[s] = bf16r(x)
        yb_v[s] = bf16r(y)
        zb_v[s] = bf16r(z)
        return 0

    lax.fori_loop(0, N // 16, r2row, 0)

    def scan_row(i, wcur):
        def scan_grp(jj, wc):
            cvec = cl_v[i, pl.ds(16 * jj, 16)]
            m = cvec > 0
            ids = (i * 256 + 16 * jj) + iota16
            plsc.store_compressed(wl_v.at[pl.ds(wc, 16)], ids, mask=m)
            npos = plsc.all_reduce_population_count(m)
            return wc + npos[0]

        return lax.fori_loop(0, 16, scan_grp, wcur)

    nwork = lax.fori_loop(0, BAND, scan_row, 0)

    def visit(e, cur):
        wid = wl_v[pl.ds(e, 16)][0]
        i = wid // 256
        jb = (wid % 256) * 16
        gi = r0 + i
        giv = zeros16 + gi
        xi = x_v[pl.ds(gi, 16)][0]
        yi = y_v[pl.ds(gi, 16)][0]
        zi = z_v[pl.ds(gi, 16)][0]
        r2i = r2_v[pl.ds(gi, 16)][0]
        xbi = xb_v[pl.ds(gi, 16)][0]
        ybi = yb_v[pl.ds(gi, 16)][0]
        zbi = zb_v[pl.ds(gi, 16)][0]
        xj = x_v[pl.ds(jb, 16)]
        yj = y_v[pl.ds(jb, 16)]
        zj = z_v[pl.ds(jb, 16)]
        r2j = r2_v[pl.ds(jb, 16)]
        dot = xbi * xb_v[pl.ds(jb, 16)] + ybi * yb_v[pl.ds(jb, 16)] \
            + zbi * zb_v[pl.ds(jb, 16)]
        d2a = (r2i + r2j) - 2.0 * dot
        jv = jb + iota16
        m = (d2a < CUT * CUT) & (jv != giv)
        dx = xi - xj
        dy = yi - yj
        dz = zi - zj
        d2b = dx * dx + dy * dy + dz * dz
        plsc.store_compressed(sndl_v.at[pl.ds(cur, 16)], jv, mask=m)
        plsc.store_compressed(rcvl_v.at[pl.ds(cur, 16)], giv, mask=m)
        plsc.store_compressed(d2l_v.at[pl.ds(cur, 16)], d2b, mask=m)
        npos = plsc.all_reduce_population_count(m)
        return cur + npos[0]

    ecnt = lax.fori_loop(0, nwork, visit, 0)

    def padrow(k, _):
        s = pl.ds(ecnt + 16 * k, 16)
        sndl_v[s] = zeros16
        rcvl_v[s] = zeros16 + N
        d2l_v[s] = jnp.zeros((16,), jnp.float32) + 1.0
        return 0

    lax.fori_loop(0, CHUNK // 16, padrow, 0)

    ew_v[...] = zeros16 + ecnt
    pltpu.sync_copy(ew_v, ecnt_ref.at[tile])
    pltpu.sync_copy(sndl_v, snd_ref.at[pl.ds(tile * SLAB, SLAB)])
    pltpu.sync_copy(rcvl_v, rcv_ref.at[pl.ds(tile * SLAB, SLAB)])
    pltpu.sync_copy(d2l_v, d2_ref.at[pl.ds(tile * SLAB, SLAB)])


def _sc_edges(xs, ys, zs, cnts):
    mesh = plsc.VectorSubcoreMesh(core_axis_name="c", subcore_axis_name="s")
    f = pl.kernel(
        _edges_body,
        mesh=mesh,
        compiler_params=pltpu.CompilerParams(needs_layout_passes=False),
        out_type=[
            jax.ShapeDtypeStruct((ECAP,), jnp.int32),
            jax.ShapeDtypeStruct((ECAP,), jnp.int32),
            jax.ShapeDtypeStruct((ECAP,), jnp.float32),
            jax.ShapeDtypeStruct((N_TILES, 16), jnp.int32),
        ],
        scratch_types=[
            pltpu.VMEM((N + 16,), jnp.float32),
            pltpu.VMEM((N + 16,), jnp.float32),
            pltpu.VMEM((N + 16,), jnp.float32),
            pltpu.VMEM((N + 16,), jnp.float32),
            pltpu.VMEM((N + 16,), jnp.float32),
            pltpu.VMEM((N + 16,), jnp.float32),
            pltpu.VMEM((N + 16,), jnp.float32),
            pltpu.VMEM((BAND, 256), jnp.int32),
            pltpu.VMEM((NWORK + 16,), jnp.int32),
            pltpu.VMEM((SLAB,), jnp.int32),
            pltpu.VMEM((SLAB,), jnp.int32),
            pltpu.VMEM((SLAB,), jnp.float32),
            pltpu.VMEM((16,), jnp.int32),
            pltpu.SemaphoreType.DMA,
        ],
    )
    return f(xs, ys, zs, cnts)


def _bf16_hi(x):
    b = lax.bitcast_convert_type(x, jnp.int32)
    return (b + 0x7FFF + ((b >> 16) & 1)) & (-65536)


def _eproj_body(d2_ref, snd_ref, we0_ref, we1_ref, we2_ref, o0_ref, o1_ref, o2_ref):
    d = jnp.sqrt(d2_ref[...] + 1e-12)
    valid = (snd_ref[...] < N).astype(jnp.float32)
    mu = (CUT / (NB - 1)) * lax.broadcasted_iota(jnp.int32, (1, NB), 1).astype(jnp.float32)
    sigma = CUT / NB
    e16 = jnp.exp(-((d - mu) ** 2) / (2.0 * sigma * sigma)) * valid
    for we_ref, o_ref in ((we0_ref, o0_ref), (we1_ref, o1_ref), (we2_ref, o2_ref)):
        lo = _bf16_hi(jnp.dot(e16, we_ref[:, :D // 2],
                              preferred_element_type=jnp.float32,
                              precision=lax.Precision.HIGHEST))
        hi = _bf16_hi(jnp.dot(e16, we_ref[:, D // 2:],
                              preferred_element_type=jnp.float32,
                              precision=lax.Precision.HIGHEST))
        o_ref[...] = hi | lax.shift_right_logical(lo, 16)


def _eproj_all(d2, snd, W_e_0, W_e_1, W_e_2):
    rows = 2048
    grid = ECAP // rows
    d2 = d2.reshape(ECAP, 1)
    s2 = snd.reshape(ECAP, 1)
    out = jax.ShapeDtypeStruct((ECAP, D // 2), jnp.int32)
    return pl.pallas_call(
        _eproj_body,
        grid=(grid,),
        in_specs=[
            pl.BlockSpec((rows, 1), lambda i: (i, 0)),
            pl.BlockSpec((rows, 1), lambda i: (i, 0)),
            pl.BlockSpec((NB, D), lambda i: (0, 0)),
            pl.BlockSpec((NB, D), lambda i: (0, 0)),
            pl.BlockSpec((NB, D), lambda i: (0, 0)),
        ],
        out_specs=[pl.BlockSpec((rows, D // 2), lambda i: (i, 0))] * 3,
        out_shape=[out, out, out],
    )(d2, s2, W_e_0, W_e_1, W_e_2)


BAND = N // N_TILES


def _msg_body(h_ref, ep_ref, snd_ref, rcv_ref, ecnt_ref, out_ref,
              bnd_v, snda_v, rcva_v, snd_v0, snd_v1, rows_v0, rows_v1,
              ep_v0, ep_v1, agg_v, semg0, semg1, seme0, seme1):
    cid = lax.axis_index("c")
    sid = lax.axis_index("s")
    band = cid * 16 + sid

    pltpu.sync_copy(ecnt_ref.at[band], bnd_v)
    ecnt = bnd_v[pl.ds(0, 16)][0]
    estart = band * SLAB
    nch = (ecnt + CHUNK - 1) // CHUNK

    pltpu.sync_copy(snd_ref.at[pl.ds(estart, SLAB)], snda_v)
    pltpu.sync_copy(rcv_ref.at[pl.ds(estart, SLAB)], rcva_v.at[pl.ds(0, SLAB)])

    zero16f = jnp.zeros((16,), jnp.float32)

    def zrow(r, _):
        for c in range(8):
            agg_v[r, pl.ds(16 * c, 16)] = zero16f
        return 0

    lax.fori_loop(0, BAND + 1, zrow, 0)

    iota16 = lax.iota(jnp.int32, 16)
    base = band * BAND
    slots = ((snd_v0, rows_v0, ep_v0, semg0, seme0),
             (snd_v1, rows_v1, ep_v1, semg1, seme1))

    def load(g, slot):
        snd_v, rows_v, ep_v, semg, seme = slots[slot]
        loc = g * CHUNK
        for k in range(8):
            snd_v[pl.ds(16 * k, 16)] = snda_v[pl.ds(loc + 16 * k, 16)]
        pltpu.async_copy(h_ref.at[snd_v], rows_v, semg)
        pltpu.async_copy(ep_ref.at[pl.ds(estart + loc, CHUNK)], ep_v, seme)

    def compute(g, slot):
        snd_v, rows_v, ep_v, semg, seme = slots[slot]
        pltpu.make_async_copy(h_ref.at[snd_v], rows_v, semg).wait()
        pltpu.make_async_copy(ep_ref.at[pl.ds(0, CHUNK)], ep_v, seme).wait()
        loc = g * CHUNK

        def edge16(q, _):
            v = rcva_v[pl.ds(loc + q * 16, 16)] - base
            rowsel = jnp.where((v >= 0) & (v < BAND), v, BAND)
            for u in range(16):
                e = q * 16 + u
                rowv = jnp.full((16,), rowsel[u], jnp.int32)
                for c in range(4):
                    bits = ep_v[e, pl.ds(16 * c, 16)]
                    lo = plsc.bitcast(bits << 16, jnp.float32)
                    hi = plsc.bitcast(bits & (-65536), jnp.float32)
                    sl0 = pl.ds(32 * c, 16)
                    sl1 = pl.ds(32 * c + 16, 16)
                    plsc.addupdate_scatter(agg_v, [rowv, iota16 + (32 * c)],
                                           rows_v[e, sl0] * lo)
                    plsc.addupdate_scatter(agg_v, [rowv, iota16 + (32 * c + 16)],
                                           rows_v[e, sl1] * hi)
            return 0

        lax.fori_loop(0, CHUNK // 16, edge16, 0)

    @pl.when(nch > 0)
    def _():
        load(0, 0)

    def chunk(g, _):
        slot = lax.rem(g, 2)

        @pl.when(g + 1 < nch)
        def _():
            ns = lax.rem(g + 1, 2)
            lax.cond(ns == 0, lambda: load(g + 1, 0), lambda: load(g + 1, 1))

        lax.cond(slot == 0, lambda: compute(g, 0), lambda: compute(g, 1))
        return 0

    lax.fori_loop(0, nch, chunk, 0)
    pltpu.sync_copy(agg_v.at[pl.ds(0, BAND)], out_ref.at[pl.ds(band * BAND, BAND)])


def _sc_messages(h, eproj, snd, rcv, ecnt):
    mesh = plsc.VectorSubcoreMesh(core_axis_name="c", subcore_axis_name="s")
    f = pl.kernel(
        _msg_body,
        mesh=mesh,
        compiler_params=pltpu.CompilerParams(needs_layout_passes=False),
        out_type=jax.ShapeDtypeStruct((N, D), jnp.float32),
        scratch_types=[
            pltpu.VMEM((16,), jnp.int32),
            pltpu.VMEM((SLAB,), jnp.int32),
            pltpu.VMEM((SLAB + 16,), jnp.int32),
            pltpu.VMEM((CHUNK,), jnp.int32),
            pltpu.VMEM((CHUNK,), jnp.int32),
            pltpu.VMEM((CHUNK, D), jnp.float32),
            pltpu.VMEM((CHUNK, D), jnp.float32),
            pltpu.VMEM((CHUNK, D // 2), jnp.int32),
            pltpu.VMEM((CHUNK, D // 2), jnp.int32),
            pltpu.VMEM((BAND + 1, D), jnp.float32),
            pltpu.SemaphoreType.DMA,
            pltpu.SemaphoreType.DMA,
            pltpu.SemaphoreType.DMA,
            pltpu.SemaphoreType.DMA,
        ],
    )
    return f(h, eproj, snd, rcv, ecnt)


def _update_body(h_ref, p0_ref, wmsg_ref, wuh_ref, wua_ref, o_ref):
    h = h_ref[...]
    agg = p0_ref[...]
    pre = jnp.dot(agg, wmsg_ref[...], preferred_element_type=jnp.float32,
                  precision=lax.Precision.HIGHEST)
    z = (jnp.dot(h, wuh_ref[...], preferred_element_type=jnp.float32,
                 precision=lax.Precision.HIGHEST)
         + jnp.dot(pre, wua_ref[...], preferred_element_type=jnp.float32,
                   precision=lax.Precision.HIGHEST))
    o_ref[...] = h + jnp.maximum(z, 0.0)


def _node_update(h, p0, W_msg, W_u):
    rows = 512
    return pl.pallas_call(
        _update_body,
        grid=(N // rows,),
        in_specs=[
            pl.BlockSpec((rows, D), lambda i: (i, 0)),
            pl.BlockSpec((rows, D), lambda i: (i, 0)),
            pl.BlockSpec((D, D), lambda i: (0, 0)),
            pl.BlockSpec((D, D), lambda i: (0, 0)),
            pl.BlockSpec((D, D), lambda i: (0, 0)),
        ],
        out_specs=pl.BlockSpec((rows, D), lambda i: (i, 0)),
        out_shape=jax.ShapeDtypeStruct((N, D), jnp.float32),
    )(h, p0, W_msg, W_u[:D], W_u[D:])


def kernel(r, node_embed, W_e_0, W_msg_0, W_u_0, W_e_1, W_msg_1, W_u_1,
           W_e_2, W_msg_2, W_u_2):
    rt = r.T
    p = jnp.repeat(jnp.eye(N // 16, dtype=jnp.float32), 16, axis=0)
    cnts = _prefilter(r, rt, p)
    snd, rcv, d2, ecnt = _sc_edges(rt[0], rt[1], rt[2], cnts)
    lo_idx = [32 * (k // 16) + (k % 16) for k in range(D // 2)]
    hi_idx = [c + 16 for c in lo_idx]
    perm = jnp.array(lo_idx + hi_idx, dtype=jnp.int32)
    ep0, ep1, ep2 = _eproj_all(d2, snd, W_e_0[:, perm], W_e_1[:, perm],
                               W_e_2[:, perm])

    h = node_embed
    for eproj, W_msg, W_u in ((ep0, W_msg_0, W_u_0), (ep1, W_msg_1, W_u_1),
                              (ep2, W_msg_2, W_u_2)):
        agg = _sc_messages(h, eproj, snd, rcv, ecnt)
        h = _node_update(h, agg, W_msg, W_u)
    return h


# speedup vs baseline: 1.2712x; 1.2712x over previous
import functools

import jax
import jax.numpy as jnp
from jax import lax
from jax.experimental import pallas as pl
from jax.experimental.pallas import tpu as pltpu
from jax.experimental.pallas import tpu_sc as plsc

N = 4096
D = 128
CUT = 0.125
NB = 16
OCC = 163840

N_TILES = 32
CHUNK = 128
BAND = N // N_TILES
SLAB = 6272
ECAP = N_TILES * SLAB
NWORK = 6144
C2EPS = CUT * CUT + 1e-4


def _prefilter_body(rblk_ref, rt_ref, p_ref, o_ref):
    rblk = rblk_ref[...]
    rt = rt_ref[...]
    r2row = jnp.sum(rblk * rblk, axis=1, keepdims=True)
    r2col = jnp.sum(rt * rt, axis=0, keepdims=True)
    dot = jnp.dot(rblk, rt, preferred_element_type=jnp.float32)
    dist2 = (r2row + r2col) - 2.0 * dot
    m = (dist2 < C2EPS).astype(jnp.float32)
    cnt = jnp.dot(m, p_ref[...], preferred_element_type=jnp.float32)
    o_ref[...] = cnt.astype(jnp.int32)


def _prefilter(r, rt, p):
    rows = 256
    return pl.pallas_call(
        _prefilter_body,
        grid=(N // rows,),
        in_specs=[
            pl.BlockSpec((rows, 3), lambda i: (i, 0)),
            pl.BlockSpec((3, N), lambda i: (0, 0)),
            pl.BlockSpec((N, N // 16), lambda i: (0, 0)),
        ],
        out_specs=pl.BlockSpec((rows, N // 16), lambda i: (i, 0)),
        out_shape=jax.ShapeDtypeStruct((N, N // 16), jnp.int32),
    )(r, rt, p)


def _edges_body(xs_ref, ys_ref, zs_ref, cnts_ref, snd_ref, rcv_ref, d2_ref,
                ecnt_ref, x_v, y_v, z_v, r2_v, xb_v, yb_v, zb_v, cl_v, wl_v,
                sndl_v, rcvl_v, d2l_v, ew_v, sem):
    cid = lax.axis_index("c")
    sid = lax.axis_index("s")
    tile = cid * 16 + sid
    r0 = tile * BAND

    pltpu.sync_copy(xs_ref, x_v.at[pl.ds(0, N)])
    pltpu.sync_copy(ys_ref, y_v.at[pl.ds(0, N)])
    pltpu.sync_copy(zs_ref, z_v.at[pl.ds(0, N)])
    pltpu.sync_copy(cnts_ref.at[pl.ds(r0, BAND)], cl_v)

    zeros16 = jnp.zeros((16,), jnp.int32)
    iota16 = lax.iota(jnp.int32, 16)

    def bf16r(v):
        b = plsc.bitcast(v, jnp.int32)
        b = (b + 0x7FFF + ((b >> 16) & 1)) & ~0xFFFF
        return plsc.bitcast(b, jnp.float32)

    def r2row(q, _):
        s = pl.ds(16 * q, 16)
        x = x_v[s]
        y = y_v[s]
        z = z_v[s]
        r2_v[s] = x * x + y * y + z * z
        xb_v[s] = bf16r(x)
        yb_v[s] = bf16r(y)
        zb_v[s] = bf16r(z)
        return 0

    lax.fori_loop(0, N // 16, r2row, 0)

    def scan_row(i, wcur):
        def scan_grp(jj, wc):
            cvec = cl_v[i, pl.ds(16 * jj, 16)]
            m = cvec > 0
            ids = (i * 256 + 16 * jj) + iota16
            plsc.store_compressed(wl_v.at[pl.ds(wc, 16)], ids, mask=m)
            npos = plsc.all_reduce_population_count(m)
            return wc + npos[0]

        return lax.fori_loop(0, 16, scan_grp, wcur)

    nwork = lax.fori_loop(0, BAND, scan_row, 0)

    def visit(e, cur):
        wid = wl_v[pl.ds(e, 16)][0]
        i = wid // 256
        jb = (wid % 256) * 16
        gi = r0 + i
        giv = zeros16 + gi
        xi = x_v[pl.ds(gi, 16)][0]
        yi = y_v[pl.ds(gi, 16)][0]
        zi = z_v[pl.ds(gi, 16)][0]
        r2i = r2_v[pl.ds(gi, 16)][0]
        xbi = xb_v[pl.ds(gi, 16)][0]
        ybi = yb_v[pl.ds(gi, 16)][0]
        zbi = zb_v[pl.ds(gi, 16)][0]
        xj = x_v[pl.ds(jb, 16)]
        yj = y_v[pl.ds(jb, 16)]
        zj = z_v[pl.ds(jb, 16)]
        r2j = r2_v[pl.ds(jb, 16)]
        dot = xbi * xb_v[pl.ds(jb, 16)] + ybi * yb_v[pl.ds(jb, 16)] \
            + zbi * zb_v[pl.ds(jb, 16)]
        d2a = (r2i + r2j) - 2.0 * dot
        jv = jb + iota16
        m = (d2a < CUT * CUT) & (jv != giv)
        dx = xi - xj
        dy = yi - yj
        dz = zi - zj
        d2b = dx * dx + dy * dy + dz * dz
        plsc.store_compressed(sndl_v.at[pl.ds(cur, 16)], jv, mask=m)
        plsc.store_compressed(rcvl_v.at[pl.ds(cur, 16)], giv, mask=m)
        plsc.store_compressed(d2l_v.at[pl.ds(cur, 16)], d2b, mask=m)
        npos = plsc.all_reduce_population_count(m)
        return cur + npos[0]

    ecnt = lax.fori_loop(0, nwork, visit, 0)

    def padrow(k, _):
        s = pl.ds(ecnt + 16 * k, 16)
        sndl_v[s] = zeros16
        rcvl_v[s] = zeros16 + N
        d2l_v[s] = jnp.zeros((16,), jnp.float32) + 1.0
        return 0

    lax.fori_loop(0, CHUNK // 16, padrow, 0)

    ew_v[...] = zeros16 + ecnt
    pltpu.sync_copy(ew_v, ecnt_ref.at[tile])
    pltpu.sync_copy(sndl_v, snd_ref.at[pl.ds(tile * SLAB, SLAB)])
    pltpu.sync_copy(rcvl_v, rcv_ref.at[pl.ds(tile * SLAB, SLAB)])
    pltpu.sync_copy(d2l_v, d2_ref.at[pl.ds(tile * SLAB, SLAB)])


def _sc_edges(xs, ys, zs, cnts):
    mesh = plsc.VectorSubcoreMesh(core_axis_name="c", subcore_axis_name="s")
    f = pl.kernel(
        _edges_body,
        mesh=mesh,
        compiler_params=pltpu.CompilerParams(needs_layout_passes=False),
        out_type=[
            jax.ShapeDtypeStruct((ECAP,), jnp.int32),
            jax.ShapeDtypeStruct((ECAP,), jnp.int32),
            jax.ShapeDtypeStruct((ECAP,), jnp.float32),
            jax.ShapeDtypeStruct((N_TILES, 16), jnp.int32),
        ],
        scratch_types=[
            pltpu.VMEM((N + 16,), jnp.float32),
            pltpu.VMEM((N + 16,), jnp.float32),
            pltpu.VMEM((N + 16,), jnp.float32),
            pltpu.VMEM((N + 16,), jnp.float32),
            pltpu.VMEM((N + 16,), jnp.float32),
            pltpu.VMEM((N + 16,), jnp.float32),
            pltpu.VMEM((N + 16,), jnp.float32),
            pltpu.VMEM((BAND, 256), jnp.int32),
            pltpu.VMEM((NWORK + 16,), jnp.int32),
            pltpu.VMEM((SLAB,), jnp.int32),
            pltpu.VMEM((SLAB,), jnp.int32),
            pltpu.VMEM((SLAB,), jnp.float32),
            pltpu.VMEM((16,), jnp.int32),
            pltpu.SemaphoreType.DMA,
        ],
    )
    return f(xs, ys, zs, cnts)


def _bf16_hi(x):
    b = lax.bitcast_convert_type(x, jnp.int32)
    return (b + 0x7FFF + ((b >> 16) & 1)) & (-65536)


def _eproj_body(d2_ref, snd_ref, we0_ref, we1_ref, we2_ref, o0_ref, o1_ref, o2_ref):
    d = jnp.sqrt(d2_ref[...] + 1e-12)
    valid = (snd_ref[...] < N).astype(jnp.float32)
    mu = (CUT / (NB - 1)) * lax.broadcasted_iota(jnp.int32, (1, NB), 1).astype(jnp.float32)
    sigma = CUT / NB
    e16 = jnp.exp(-((d - mu) ** 2) / (2.0 * sigma * sigma)) * valid
    for we_ref, o_ref in ((we0_ref, o0_ref), (we1_ref, o1_ref), (we2_ref, o2_ref)):
        ep = jnp.dot(e16, we_ref[...], preferred_element_type=jnp.float32)
        lo = _bf16_hi(ep[:, :D // 2])
        hi = _bf16_hi(ep[:, D // 2:])
        o_ref[...] = hi | lax.shift_right_logical(lo, 16)


def _eproj_all(d2, snd, W_e_0, W_e_1, W_e_2):
    rows = 2048
    grid = ECAP // rows
    d2 = d2.reshape(ECAP, 1)
    s2 = snd.reshape(ECAP, 1)
    out = jax.ShapeDtypeStruct((ECAP, D // 2), jnp.int32)
    return pl.pallas_call(
        _eproj_body,
        grid=(grid,),
        in_specs=[
            pl.BlockSpec((rows, 1), lambda i: (i, 0)),
            pl.BlockSpec((rows, 1), lambda i: (i, 0)),
            pl.BlockSpec((NB, D), lambda i: (0, 0)),
            pl.BlockSpec((NB, D), lambda i: (0, 0)),
            pl.BlockSpec((NB, D), lambda i: (0, 0)),
        ],
        out_specs=[pl.BlockSpec((rows, D // 2), lambda i: (i, 0))] * 3,
        out_shape=[out, out, out],
    )(d2, s2, W_e_0, W_e_1, W_e_2)


BAND = N // N_TILES


def _msg_body(h_ref, ep_ref, snd_ref, rcv_ref, ecnt_ref, out_ref,
              bnd_v, snda_v, rcva_v, snd_v0, snd_v1, rows_v0, rows_v1,
              ep_v0, ep_v1, agg_v, semg0, semg1, seme0, seme1):
    cid = lax.axis_index("c")
    sid = lax.axis_index("s")
    band = cid * 16 + sid

    pltpu.sync_copy(ecnt_ref.at[band], bnd_v)
    ecnt = bnd_v[pl.ds(0, 16)][0]
    estart = band * SLAB
    nch = (ecnt + CHUNK - 1) // CHUNK

    pltpu.sync_copy(snd_ref.at[pl.ds(estart, SLAB)], snda_v)
    pltpu.sync_copy(rcv_ref.at[pl.ds(estart, SLAB)], rcva_v.at[pl.ds(0, SLAB)])

    zero16f = jnp.zeros((16,), jnp.float32)

    def zrow(r, _):
        for c in range(8):
            agg_v[r, pl.ds(16 * c, 16)] = zero16f
        return 0

    lax.fori_loop(0, BAND + 1, zrow, 0)

    iota16 = lax.iota(jnp.int32, 16)
    base = band * BAND
    slots = ((snd_v0, rows_v0, ep_v0, semg0, seme0),
             (snd_v1, rows_v1, ep_v1, semg1, seme1))

    def load(g, slot):
        snd_v, rows_v, ep_v, semg, seme = slots[slot]
        loc = g * CHUNK
        for k in range(8):
            snd_v[pl.ds(16 * k, 16)] = snda_v[pl.ds(loc + 16 * k, 16)]
        pltpu.async_copy(h_ref.at[snd_v], rows_v, semg)
        pltpu.async_copy(ep_ref.at[pl.ds(estart + loc, CHUNK)], ep_v, seme)

    def compute(g, slot):
        snd_v, rows_v, ep_v, semg, seme = slots[slot]
        pltpu.make_async_copy(h_ref.at[snd_v], rows_v, semg).wait()
        pltpu.make_async_copy(ep_ref.at[pl.ds(0, CHUNK)], ep_v, seme).wait()
        loc = g * CHUNK

        def edge16(q, _):
            v = rcva_v[pl.ds(loc + q * 16, 16)] - base
            rowsel = jnp.where((v >= 0) & (v < BAND), v, BAND)
            for u in range(16):
                e = q * 16 + u
                rowv = jnp.full((16,), rowsel[u], jnp.int32)
                for c in range(4):
                    bits = ep_v[e, pl.ds(16 * c, 16)]
                    lo = plsc.bitcast(bits << 16, jnp.float32)
                    hi = plsc.bitcast(bits & (-65536), jnp.float32)
                    sl0 = pl.ds(32 * c, 16)
                    sl1 = pl.ds(32 * c + 16, 16)
                    plsc.addupdate_scatter(agg_v, [rowv, iota16 + (32 * c)],
                                           rows_v[e, sl0] * lo)
                    plsc.addupdate_scatter(agg_v, [rowv, iota16 + (32 * c + 16)],
                                           rows_v[e, sl1] * hi)
            return 0

        lax.fori_loop(0, CHUNK // 16, edge16, 0)

    @pl.when(nch > 0)
    def _():
        load(0, 0)

    def chunk(g, _):
        slot = lax.rem(g, 2)

        @pl.when(g + 1 < nch)
        def _():
            ns = lax.rem(g + 1, 2)
            lax.cond(ns == 0, lambda: load(g + 1, 0), lambda: load(g + 1, 1))

        lax.cond(slot == 0, lambda: compute(g, 0), lambda: compute(g, 1))
        return 0

    lax.fori_loop(0, nch, chunk, 0)
    pltpu.sync_copy(agg_v.at[pl.ds(0, BAND)], out_ref.at[pl.ds(band * BAND, BAND)])


def _sc_messages(h, eproj, snd, rcv, ecnt):
    mesh = plsc.VectorSubcoreMesh(core_axis_name="c", subcore_axis_name="s")
    f = pl.kernel(
        _msg_body,
        mesh=mesh,
        compiler_params=pltpu.CompilerParams(needs_layout_passes=False),
        out_type=jax.ShapeDtypeStruct((N, D), jnp.float32),
        scratch_types=[
            pltpu.VMEM((16,), jnp.int32),
            pltpu.VMEM((SLAB,), jnp.int32),
            pltpu.VMEM((SLAB + 16,), jnp.int32),
            pltpu.VMEM((CHUNK,), jnp.int32),
            pltpu.VMEM((CHUNK,), jnp.int32),
            pltpu.VMEM((CHUNK, D), jnp.float32),
            pltpu.VMEM((CHUNK, D), jnp.float32),
            pltpu.VMEM((CHUNK, D // 2), jnp.int32),
            pltpu.VMEM((CHUNK, D // 2), jnp.int32),
            pltpu.VMEM((BAND + 1, D), jnp.float32),
            pltpu.SemaphoreType.DMA,
            pltpu.SemaphoreType.DMA,
            pltpu.SemaphoreType.DMA,
            pltpu.SemaphoreType.DMA,
        ],
    )
    return f(h, eproj, snd, rcv, ecnt)


def _update_body(h_ref, p0_ref, wmsg_ref, wuh_ref, wua_ref, o_ref):
    h = h_ref[...]
    agg = p0_ref[...]
    pre = jnp.dot(agg, wmsg_ref[...], preferred_element_type=jnp.float32)
    z = (jnp.dot(h, wuh_ref[...], preferred_element_type=jnp.float32)
         + jnp.dot(pre, wua_ref[...], preferred_element_type=jnp.float32))
    o_ref[...] = h + jnp.maximum(z, 0.0)


def _node_update(h, p0, W_msg, W_u):
    rows = 512
    return pl.pallas_call(
        _update_body,
        grid=(N // rows,),
        in_specs=[
            pl.BlockSpec((rows, D), lambda i: (i, 0)),
            pl.BlockSpec((rows, D), lambda i: (i, 0)),
            pl.BlockSpec((D, D), lambda i: (0, 0)),
            pl.BlockSpec((D, D), lambda i: (0, 0)),
            pl.BlockSpec((D, D), lambda i: (0, 0)),
        ],
        out_specs=pl.BlockSpec((rows, D), lambda i: (i, 0)),
        out_shape=jax.ShapeDtypeStruct((N, D), jnp.float32),
    )(h, p0, W_msg, W_u[:D], W_u[D:])


def kernel(r, node_embed, W_e_0, W_msg_0, W_u_0, W_e_1, W_msg_1, W_u_1,
           W_e_2, W_msg_2, W_u_2):
    rt = r.T
    p = jnp.repeat(jnp.eye(N // 16, dtype=jnp.float32), 16, axis=0)
    cnts = _prefilter(r, rt, p)
    snd, rcv, d2, ecnt = _sc_edges(rt[0], rt[1], rt[2], cnts)
    lo_idx = [32 * (k // 16) + (k % 16) for k in range(D // 2)]
    hi_idx = [c + 16 for c in lo_idx]
    perm = jnp.array(lo_idx + hi_idx, dtype=jnp.int32)
    ep0, ep1, ep2 = _eproj_all(d2, snd, W_e_0[:, perm], W_e_1[:, perm],
                               W_e_2[:, perm])

    h = node_embed
    for eproj, W_msg, W_u in ((ep0, W_msg_0, W_u_0), (ep1, W_msg_1, W_u_1),
                              (ep2, W_msg_2, W_u_2)):
        agg = _sc_messages(h, eproj, snd, rcv, ecnt)
        h = _node_update(h, agg, W_msg, W_u)
    return h


# speedup vs baseline: 1.4125x; 1.1111x over previous
import functools

import jax
import jax.numpy as jnp
from jax import lax
from jax.experimental import pallas as pl
from jax.experimental.pallas import tpu as pltpu
from jax.experimental.pallas import tpu_sc as plsc

N = 4096
D = 128
CUT = 0.125
NB = 16
OCC = 163840

N_TILES = 32
CHUNK = 128
BAND = N // N_TILES
SLAB = 6272
ECAP = N_TILES * SLAB
NWORK = 6144
C2EPS = CUT * CUT + 1e-4


def _prefilter_body(rblk_ref, rt_ref, p_ref, o_ref):
    rblk = rblk_ref[...]
    rt = rt_ref[...]
    r2row = jnp.sum(rblk * rblk, axis=1, keepdims=True)
    r2col = jnp.sum(rt * rt, axis=0, keepdims=True)
    dot = jnp.dot(rblk, rt, preferred_element_type=jnp.float32)
    dist2 = (r2row + r2col) - 2.0 * dot
    m = (dist2 < C2EPS).astype(jnp.float32)
    cnt = jnp.dot(m, p_ref[...], preferred_element_type=jnp.float32)
    o_ref[...] = cnt.astype(jnp.int32)


def _prefilter(r, rt, p):
    rows = 256
    return pl.pallas_call(
        _prefilter_body,
        grid=(N // rows,),
        in_specs=[
            pl.BlockSpec((rows, 3), lambda i: (i, 0)),
            pl.BlockSpec((3, N), lambda i: (0, 0)),
            pl.BlockSpec((N, N // 16), lambda i: (0, 0)),
        ],
        out_specs=pl.BlockSpec((rows, N // 16), lambda i: (i, 0)),
        out_shape=jax.ShapeDtypeStruct((N, N // 16), jnp.int32),
    )(r, rt, p)


def _edges_body(xs_ref, ys_ref, zs_ref, cnts_ref, snd_ref, rcv_ref, d2_ref,
                ecnt_ref, x_v, y_v, z_v, r2_v, xb_v, yb_v, zb_v, cl_v, wl_v,
                sndl_v, rcvl_v, d2l_v, ew_v, sem):
    cid = lax.axis_index("c")
    sid = lax.axis_index("s")
    tile = cid * 16 + sid
    r0 = tile * BAND

    pltpu.sync_copy(xs_ref, x_v.at[pl.ds(0, N)])
    pltpu.sync_copy(ys_ref, y_v.at[pl.ds(0, N)])
    pltpu.sync_copy(zs_ref, z_v.at[pl.ds(0, N)])
    pltpu.sync_copy(cnts_ref.at[pl.ds(r0, BAND)], cl_v)

    zeros16 = jnp.zeros((16,), jnp.int32)
    iota16 = lax.iota(jnp.int32, 16)

    def bf16r(v):
        b = plsc.bitcast(v, jnp.int32)
        b = (b + 0x7FFF + ((b >> 16) & 1)) & ~0xFFFF
        return plsc.bitcast(b, jnp.float32)

    def r2row(q, _):
        s = pl.ds(16 * q, 16)
        x = x_v[s]
        y = y_v[s]
        z = z_v[s]
        r2_v[s] = x * x + y * y + z * z
        xb_v[s] = bf16r(x)
        yb_v[s] = bf16r(y)
        zb_v[s] = bf16r(z)
        return 0

    lax.fori_loop(0, N // 16, r2row, 0)

    def scan_row(i, wcur):
        def scan_grp(jj, wc):
            cvec = cl_v[i, pl.ds(16 * jj, 16)]
            m = cvec > 0
            ids = (i * 256 + 16 * jj) + iota16
            plsc.store_compressed(wl_v.at[pl.ds(wc, 16)], ids, mask=m)
            npos = plsc.all_reduce_population_count(m)
            return wc + npos[0]

        return lax.fori_loop(0, 16, scan_grp, wcur)

    nwork = lax.fori_loop(0, BAND, scan_row, 0)

    def visit(e, cur):
        wid = wl_v[pl.ds(e, 16)][0]
        i = wid // 256
        jb = (wid % 256) * 16
        gi = r0 + i
        giv = zeros16 + gi
        xi = x_v[pl.ds(gi, 16)][0]
        yi = y_v[pl.ds(gi, 16)][0]
        zi = z_v[pl.ds(gi, 16)][0]
        r2i = r2_v[pl.ds(gi, 16)][0]
        xbi = xb_v[pl.ds(gi, 16)][0]
        ybi = yb_v[pl.ds(gi, 16)][0]
        zbi = zb_v[pl.ds(gi, 16)][0]
        xj = x_v[pl.ds(jb, 16)]
        yj = y_v[pl.ds(jb, 16)]
        zj = z_v[pl.ds(jb, 16)]
        r2j = r2_v[pl.ds(jb, 16)]
        dot = xbi * xb_v[pl.ds(jb, 16)] + ybi * yb_v[pl.ds(jb, 16)] \
            + zbi * zb_v[pl.ds(jb, 16)]
        d2a = (r2i + r2j) - 2.0 * dot
        jv = jb + iota16
        m = (d2a < CUT * CUT) & (jv != giv)
        dx = xi - xj
        dy = yi - yj
        dz = zi - zj
        d2b = dx * dx + dy * dy + dz * dz
        plsc.store_compressed(sndl_v.at[pl.ds(cur, 16)], jv, mask=m)
        plsc.store_compressed(rcvl_v.at[pl.ds(cur, 16)], giv, mask=m)
        plsc.store_compressed(d2l_v.at[pl.ds(cur, 16)], d2b, mask=m)
        npos = plsc.all_reduce_population_count(m)
        return cur + npos[0]

    ecnt = lax.fori_loop(0, nwork, visit, 0)

    def padrow(k, _):
        s = pl.ds(ecnt + 16 * k, 16)
        sndl_v[s] = zeros16
        rcvl_v[s] = zeros16 + N
        d2l_v[s] = jnp.zeros((16,), jnp.float32) + 1.0
        return 0

    lax.fori_loop(0, CHUNK // 16, padrow, 0)

    ew_v[...] = zeros16 + ecnt
    pltpu.sync_copy(ew_v, ecnt_ref.at[tile])
    pltpu.sync_copy(sndl_v, snd_ref.at[pl.ds(tile * SLAB, SLAB)])
    pltpu.sync_copy(rcvl_v, rcv_ref.at[pl.ds(tile * SLAB, SLAB)])
    pltpu.sync_copy(d2l_v, d2_ref.at[pl.ds(tile * SLAB, SLAB)])


def _sc_edges(xs, ys, zs, cnts):
    mesh = plsc.VectorSubcoreMesh(core_axis_name="c", subcore_axis_name="s")
    f = pl.kernel(
        _edges_body,
        mesh=mesh,
        compiler_params=pltpu.CompilerParams(needs_layout_passes=False),
        out_type=[
            jax.ShapeDtypeStruct((ECAP,), jnp.int32),
            jax.ShapeDtypeStruct((ECAP,), jnp.int32),
            jax.ShapeDtypeStruct((ECAP,), jnp.float32),
            jax.ShapeDtypeStruct((N_TILES, 16), jnp.int32),
        ],
        scratch_types=[
            pltpu.VMEM((N + 16,), jnp.float32),
            pltpu.VMEM((N + 16,), jnp.float32),
            pltpu.VMEM((N + 16,), jnp.float32),
            pltpu.VMEM((N + 16,), jnp.float32),
            pltpu.VMEM((N + 16,), jnp.float32),
            pltpu.VMEM((N + 16,), jnp.float32),
            pltpu.VMEM((N + 16,), jnp.float32),
            pltpu.VMEM((BAND, 256), jnp.int32),
            pltpu.VMEM((NWORK + 16,), jnp.int32),
            pltpu.VMEM((SLAB,), jnp.int32),
            pltpu.VMEM((SLAB,), jnp.int32),
            pltpu.VMEM((SLAB,), jnp.float32),
            pltpu.VMEM((16,), jnp.int32),
            pltpu.SemaphoreType.DMA,
        ],
    )
    return f(xs, ys, zs, cnts)


def _bf16_hi(x):
    b = lax.bitcast_convert_type(x, jnp.int32)
    return (b + 0x7FFF + ((b >> 16) & 1)) & (-65536)


def _eproj_body(d2_ref, snd_ref, we0_ref, we1_ref, we2_ref, o0_ref, o1_ref, o2_ref):
    d = jnp.sqrt(d2_ref[...] + 1e-12)
    valid = (snd_ref[...] < N).astype(jnp.float32)
    mu = (CUT / (NB - 1)) * lax.broadcasted_iota(jnp.int32, (1, NB), 1).astype(jnp.float32)
    sigma = CUT / NB
    e16 = jnp.exp(-((d - mu) ** 2) / (2.0 * sigma * sigma)) * valid
    for we_ref, o_ref in ((we0_ref, o0_ref), (we1_ref, o1_ref), (we2_ref, o2_ref)):
        ep = jnp.dot(e16, we_ref[...], preferred_element_type=jnp.float32)
        lo = _bf16_hi(ep[:, :D // 2])
        hi = _bf16_hi(ep[:, D // 2:])
        o_ref[...] = hi | lax.shift_right_logical(lo, 16)


def _eproj_all(d2, snd, W_e_0, W_e_1, W_e_2):
    rows = 2048
    grid = ECAP // rows
    d2 = d2.reshape(ECAP, 1)
    s2 = snd.reshape(ECAP, 1)
    out = jax.ShapeDtypeStruct((ECAP, D // 2), jnp.int32)
    return pl.pallas_call(
        _eproj_body,
        grid=(grid,),
        in_specs=[
            pl.BlockSpec((rows, 1), lambda i: (i, 0)),
            pl.BlockSpec((rows, 1), lambda i: (i, 0)),
            pl.BlockSpec((NB, D), lambda i: (0, 0)),
            pl.BlockSpec((NB, D), lambda i: (0, 0)),
            pl.BlockSpec((NB, D), lambda i: (0, 0)),
        ],
        out_specs=[pl.BlockSpec((rows, D // 2), lambda i: (i, 0))] * 3,
        out_shape=[out, out, out],
    )(d2, s2, W_e_0, W_e_1, W_e_2)


BAND = N // N_TILES


def _msg_body(h_ref, ep_ref, snd_ref, rcv_ref, ecnt_ref, out_ref,
              bnd_v, snda_v, rcva_v, snd_v0, snd_v1, rows_v0, rows_v1,
              ep_v0, ep_v1, agg_v, semg0, semg1, seme0, seme1):
    cid = lax.axis_index("c")
    sid = lax.axis_index("s")
    band = cid * 16 + sid

    pltpu.sync_copy(ecnt_ref.at[band], bnd_v)
    ecnt = bnd_v[pl.ds(0, 16)][0]
    estart = band * SLAB
    nch = (ecnt + CHUNK - 1) // CHUNK

    pltpu.sync_copy(snd_ref.at[pl.ds(estart, SLAB)], snda_v)
    pltpu.sync_copy(rcv_ref.at[pl.ds(estart, SLAB)], rcva_v.at[pl.ds(0, SLAB)])

    zero16f = jnp.zeros((16,), jnp.float32)

    def zrow(r, _):
        for c in range(8):
            agg_v[r, pl.ds(16 * c, 16)] = zero16f
        return 0

    lax.fori_loop(0, BAND + 1, zrow, 0)

    iota16 = lax.iota(jnp.int32, 16)
    base = band * BAND
    slots = ((snd_v0, rows_v0, ep_v0, semg0, seme0),
             (snd_v1, rows_v1, ep_v1, semg1, seme1))

    def load(g, slot):
        snd_v, rows_v, ep_v, semg, seme = slots[slot]
        loc = g * CHUNK
        for k in range(8):
            snd_v[pl.ds(16 * k, 16)] = snda_v[pl.ds(loc + 16 * k, 16)]
        pltpu.async_copy(h_ref.at[snd_v], rows_v, semg)
        pltpu.async_copy(ep_ref.at[pl.ds(estart + loc, CHUNK)], ep_v, seme)

    def compute(g, slot):
        snd_v, rows_v, ep_v, semg, seme = slots[slot]
        pltpu.make_async_copy(h_ref.at[snd_v], rows_v, semg).wait()
        pltpu.make_async_copy(ep_ref.at[pl.ds(0, CHUNK)], ep_v, seme).wait()
        loc = g * CHUNK

        @plsc.parallel_loop(0, CHUNK // 16)
        def edge16(q):
            v = rcva_v[pl.ds(loc + q * 16, 16)] - base
            rowsel = jnp.where((v >= 0) & (v < BAND), v, BAND)
            for u in range(16):
                e = q * 16 + u
                rowv = jnp.full((16,), rowsel[u], jnp.int32)
                for c in range(4):
                    bits = ep_v[e, pl.ds(16 * c, 16)]
                    lo = plsc.bitcast(bits << 16, jnp.float32)
                    hi = plsc.bitcast(bits & (-65536), jnp.float32)
                    sl0 = pl.ds(32 * c, 16)
                    sl1 = pl.ds(32 * c + 16, 16)
                    plsc.addupdate_scatter(agg_v, [rowv, iota16 + (32 * c)],
                                           rows_v[e, sl0] * lo)
                    plsc.addupdate_scatter(agg_v, [rowv, iota16 + (32 * c + 16)],
                                           rows_v[e, sl1] * hi)

    @pl.when(nch > 0)
    def _():
        load(0, 0)

    def chunk(g, _):
        slot = lax.rem(g, 2)

        @pl.when(g + 1 < nch)
        def _():
            ns = lax.rem(g + 1, 2)
            lax.cond(ns == 0, lambda: load(g + 1, 0), lambda: load(g + 1, 1))

        lax.cond(slot == 0, lambda: compute(g, 0), lambda: compute(g, 1))
        return 0

    lax.fori_loop(0, nch, chunk, 0)
    pltpu.sync_copy(agg_v.at[pl.ds(0, BAND)], out_ref.at[pl.ds(band * BAND, BAND)])


def _sc_messages(h, eproj, snd, rcv, ecnt):
    mesh = plsc.VectorSubcoreMesh(core_axis_name="c", subcore_axis_name="s")
    f = pl.kernel(
        _msg_body,
        mesh=mesh,
        compiler_params=pltpu.CompilerParams(needs_layout_passes=False),
        out_type=jax.ShapeDtypeStruct((N, D), jnp.float32),
        scratch_types=[
            pltpu.VMEM((16,), jnp.int32),
            pltpu.VMEM((SLAB,), jnp.int32),
            pltpu.VMEM((SLAB + 16,), jnp.int32),
            pltpu.VMEM((CHUNK,), jnp.int32),
            pltpu.VMEM((CHUNK,), jnp.int32),
            pltpu.VMEM((CHUNK, D), jnp.float32),
            pltpu.VMEM((CHUNK, D), jnp.float32),
            pltpu.VMEM((CHUNK, D // 2), jnp.int32),
            pltpu.VMEM((CHUNK, D // 2), jnp.int32),
            pltpu.VMEM((BAND + 1, D), jnp.float32),
            pltpu.SemaphoreType.DMA,
            pltpu.SemaphoreType.DMA,
            pltpu.SemaphoreType.DMA,
            pltpu.SemaphoreType.DMA,
        ],
    )
    return f(h, eproj, snd, rcv, ecnt)


def _update_body(h_ref, p0_ref, wmsg_ref, wuh_ref, wua_ref, o_ref):
    h = h_ref[...]
    agg = p0_ref[...]
    pre = jnp.dot(agg, wmsg_ref[...], preferred_element_type=jnp.float32)
    z = (jnp.dot(h, wuh_ref[...], preferred_element_type=jnp.float32)
         + jnp.dot(pre, wua_ref[...], preferred_element_type=jnp.float32))
    o_ref[...] = h + jnp.maximum(z, 0.0)


def _node_update(h, p0, W_msg, W_u):
    rows = 512
    return pl.pallas_call(
        _update_body,
        grid=(N // rows,),
        in_specs=[
            pl.BlockSpec((rows, D), lambda i: (i, 0)),
            pl.BlockSpec((rows, D), lambda i: (i, 0)),
            pl.BlockSpec((D, D), lambda i: (0, 0)),
            pl.BlockSpec((D, D), lambda i: (0, 0)),
            pl.BlockSpec((D, D), lambda i: (0, 0)),
        ],
        out_specs=pl.BlockSpec((rows, D), lambda i: (i, 0)),
        out_shape=jax.ShapeDtypeStruct((N, D), jnp.float32),
    )(h, p0, W_msg, W_u[:D], W_u[D:])


def kernel(r, node_embed, W_e_0, W_msg_0, W_u_0, W_e_1, W_msg_1, W_u_1,
           W_e_2, W_msg_2, W_u_2):
    rt = r.T
    p = jnp.repeat(jnp.eye(N // 16, dtype=jnp.float32), 16, axis=0)
    cnts = _prefilter(r, rt, p)
    snd, rcv, d2, ecnt = _sc_edges(rt[0], rt[1], rt[2], cnts)
    lo_idx = [32 * (k // 16) + (k % 16) for k in range(D // 2)]
    hi_idx = [c + 16 for c in lo_idx]
    perm = jnp.array(lo_idx + hi_idx, dtype=jnp.int32)
    ep0, ep1, ep2 = _eproj_all(d2, snd, W_e_0[:, perm], W_e_1[:, perm],
                               W_e_2[:, perm])

    h = node_embed
    for eproj, W_msg, W_u in ((ep0, W_msg_0, W_u_0), (ep1, W_msg_1, W_u_1),
                              (ep2, W_msg_2, W_u_2)):
        agg = _sc_messages(h, eproj, snd, rcv, ecnt)
        h = _node_update(h, agg, W_msg, W_u)
    return h


# speedup vs baseline: 1.5572x; 1.1025x over previous
import functools

import jax
import jax.numpy as jnp
from jax import lax
from jax.experimental import pallas as pl
from jax.experimental.pallas import tpu as pltpu
from jax.experimental.pallas import tpu_sc as plsc

N = 4096
D = 128
CUT = 0.125
NB = 16
OCC = 163840

N_TILES = 32
CHUNK = 128
BAND = N // N_TILES
SLAB = 6272
ECAP = N_TILES * SLAB
NWORK = 6144
C2EPS = CUT * CUT + 1e-4


def _prefilter_body(rblk_ref, rt_ref, p_ref, o_ref):
    rblk = rblk_ref[...]
    rt = rt_ref[...]
    r2row = jnp.sum(rblk * rblk, axis=1, keepdims=True)
    r2col = jnp.sum(rt * rt, axis=0, keepdims=True)
    dot = jnp.dot(rblk, rt, preferred_element_type=jnp.float32)
    dist2 = (r2row + r2col) - 2.0 * dot
    m = (dist2 < C2EPS).astype(jnp.float32)
    cnt = jnp.dot(m, p_ref[...], preferred_element_type=jnp.float32)
    o_ref[...] = cnt.astype(jnp.int32)


def _prefilter(r, rt, p):
    rows = 256
    return pl.pallas_call(
        _prefilter_body,
        grid=(N // rows,),
        in_specs=[
            pl.BlockSpec((rows, 3), lambda i: (i, 0)),
            pl.BlockSpec((3, N), lambda i: (0, 0)),
            pl.BlockSpec((N, N // 16), lambda i: (0, 0)),
        ],
        out_specs=pl.BlockSpec((rows, N // 16), lambda i: (i, 0)),
        out_shape=jax.ShapeDtypeStruct((N, N // 16), jnp.int32),
    )(r, rt, p)


def _edges_body(xs_ref, ys_ref, zs_ref, cnts_ref, snd_ref, rcv_ref, d2_ref,
                ecnt_ref, x_v, y_v, z_v, r2_v, xb_v, yb_v, zb_v, cl_v, wl_v,
                sndl_v, rcvl_v, d2l_v, ew_v, sem):
    cid = lax.axis_index("c")
    sid = lax.axis_index("s")
    tile = cid * 16 + sid
    r0 = tile * BAND

    pltpu.sync_copy(xs_ref, x_v.at[pl.ds(0, N)])
    pltpu.sync_copy(ys_ref, y_v.at[pl.ds(0, N)])
    pltpu.sync_copy(zs_ref, z_v.at[pl.ds(0, N)])
    pltpu.sync_copy(cnts_ref.at[pl.ds(r0, BAND)], cl_v)

    zeros16 = jnp.zeros((16,), jnp.int32)
    iota16 = lax.iota(jnp.int32, 16)

    def bf16r(v):
        b = plsc.bitcast(v, jnp.int32)
        b = (b + 0x7FFF + ((b >> 16) & 1)) & ~0xFFFF
        return plsc.bitcast(b, jnp.float32)

    def r2row(q, _):
        s = pl.ds(16 * q, 16)
        x = x_v[s]
        y = y_v[s]
        z = z_v[s]
        r2_v[s] = x * x + y * y + z * z
        xb_v[s] = bf16r(x)
        yb_v[s] = bf16r(y)
        zb_v[s] = bf16r(z)
        return 0

    lax.fori_loop(0, N // 16, r2row, 0)

    @plsc.parallel_loop(0, BAND * 16, carry=jnp.int32(0))
    def scan_grp(q, wc):
        i = q // 16
        jj = q % 16
        cvec = cl_v[i, pl.ds(16 * jj, 16)]
        m = cvec > 0
        ids = (i * 256 + 16 * jj) + iota16
        plsc.store_compressed(wl_v.at[pl.ds(wc, 16)], ids, mask=m)
        npos = plsc.all_reduce_population_count(m)
        return wc + npos[0]

    nwork = scan_grp

    @plsc.parallel_loop(0, nwork, carry=jnp.int32(0))
    def visit(e, cur):
        wid = wl_v[pl.ds(e, 16)][0]
        i = wid // 256
        jb = (wid % 256) * 16
        gi = r0 + i
        giv = zeros16 + gi
        xi = x_v[pl.ds(gi, 16)][0]
        yi = y_v[pl.ds(gi, 16)][0]
        zi = z_v[pl.ds(gi, 16)][0]
        r2i = r2_v[pl.ds(gi, 16)][0]
        xbi = xb_v[pl.ds(gi, 16)][0]
        ybi = yb_v[pl.ds(gi, 16)][0]
        zbi = zb_v[pl.ds(gi, 16)][0]
        xj = x_v[pl.ds(jb, 16)]
        yj = y_v[pl.ds(jb, 16)]
        zj = z_v[pl.ds(jb, 16)]
        r2j = r2_v[pl.ds(jb, 16)]
        dot = xbi * xb_v[pl.ds(jb, 16)] + ybi * yb_v[pl.ds(jb, 16)] \
            + zbi * zb_v[pl.ds(jb, 16)]
        d2a = (r2i + r2j) - 2.0 * dot
        jv = jb + iota16
        m = (d2a < CUT * CUT) & (jv != giv)
        dx = xi - xj
        dy = yi - yj
        dz = zi - zj
        d2b = dx * dx + dy * dy + dz * dz
        plsc.store_compressed(sndl_v.at[pl.ds(cur, 16)], jv, mask=m)
        plsc.store_compressed(rcvl_v.at[pl.ds(cur, 16)], giv, mask=m)
        plsc.store_compressed(d2l_v.at[pl.ds(cur, 16)], d2b, mask=m)
        npos = plsc.all_reduce_population_count(m)
        return cur + npos[0]

    ecnt = visit

    def padrow(k, _):
        s = pl.ds(ecnt + 16 * k, 16)
        sndl_v[s] = zeros16
        rcvl_v[s] = zeros16 + N
        d2l_v[s] = jnp.zeros((16,), jnp.float32) + 1.0
        return 0

    lax.fori_loop(0, CHUNK // 16, padrow, 0)

    ew_v[...] = zeros16 + ecnt
    pltpu.sync_copy(ew_v, ecnt_ref.at[tile])
    pltpu.sync_copy(sndl_v, snd_ref.at[pl.ds(tile * SLAB, SLAB)])
    pltpu.sync_copy(rcvl_v, rcv_ref.at[pl.ds(tile * SLAB, SLAB)])
    pltpu.sync_copy(d2l_v, d2_ref.at[pl.ds(tile * SLAB, SLAB)])


def _sc_edges(xs, ys, zs, cnts):
    mesh = plsc.VectorSubcoreMesh(core_axis_name="c", subcore_axis_name="s")
    f = pl.kernel(
        _edges_body,
        mesh=mesh,
        compiler_params=pltpu.CompilerParams(needs_layout_passes=False),
        out_type=[
            jax.ShapeDtypeStruct((ECAP,), jnp.int32),
            jax.ShapeDtypeStruct((ECAP,), jnp.int32),
            jax.ShapeDtypeStruct((ECAP,), jnp.float32),
            jax.ShapeDtypeStruct((N_TILES, 16), jnp.int32),
        ],
        scratch_types=[
            pltpu.VMEM((N + 16,), jnp.float32),
            pltpu.VMEM((N + 16,), jnp.float32),
            pltpu.VMEM((N + 16,), jnp.float32),
            pltpu.VMEM((N + 16,), jnp.float32),
            pltpu.VMEM((N + 16,), jnp.float32),
            pltpu.VMEM((N + 16,), jnp.float32),
            pltpu.VMEM((N + 16,), jnp.float32),
            pltpu.VMEM((BAND, 256), jnp.int32),
            pltpu.VMEM((NWORK + 16,), jnp.int32),
            pltpu.VMEM((SLAB,), jnp.int32),
            pltpu.VMEM((SLAB,), jnp.int32),
            pltpu.VMEM((SLAB,), jnp.float32),
            pltpu.VMEM((16,), jnp.int32),
            pltpu.SemaphoreType.DMA,
        ],
    )
    return f(xs, ys, zs, cnts)


def _bf16_hi(x):
    b = lax.bitcast_convert_type(x, jnp.int32)
    return (b + 0x7FFF + ((b >> 16) & 1)) & (-65536)


def _eproj_body(d2_ref, snd_ref, we0_ref, we1_ref, we2_ref, o0_ref, o1_ref, o2_ref):
    d = jnp.sqrt(d2_ref[...] + 1e-12)
    valid = (snd_ref[...] < N).astype(jnp.float32)
    mu = (CUT / (NB - 1)) * lax.broadcasted_iota(jnp.int32, (1, NB), 1).astype(jnp.float32)
    sigma = CUT / NB
    e16 = jnp.exp(-((d - mu) ** 2) / (2.0 * sigma * sigma)) * valid
    for we_ref, o_ref in ((we0_ref, o0_ref), (we1_ref, o1_ref), (we2_ref, o2_ref)):
        ep = jnp.dot(e16, we_ref[...], preferred_element_type=jnp.float32)
        lo = _bf16_hi(ep[:, :D // 2])
        hi = _bf16_hi(ep[:, D // 2:])
        o_ref[...] = hi | lax.shift_right_logical(lo, 16)


def _eproj_all(d2, snd, W_e_0, W_e_1, W_e_2):
    rows = 2048
    grid = ECAP // rows
    d2 = d2.reshape(ECAP, 1)
    s2 = snd.reshape(ECAP, 1)
    out = jax.ShapeDtypeStruct((ECAP, D // 2), jnp.int32)
    return pl.pallas_call(
        _eproj_body,
        grid=(grid,),
        in_specs=[
            pl.BlockSpec((rows, 1), lambda i: (i, 0)),
            pl.BlockSpec((rows, 1), lambda i: (i, 0)),
            pl.BlockSpec((NB, D), lambda i: (0, 0)),
            pl.BlockSpec((NB, D), lambda i: (0, 0)),
            pl.BlockSpec((NB, D), lambda i: (0, 0)),
        ],
        out_specs=[pl.BlockSpec((rows, D // 2), lambda i: (i, 0))] * 3,
        out_shape=[out, out, out],
    )(d2, s2, W_e_0, W_e_1, W_e_2)


BAND = N // N_TILES


def _msg_body(h_ref, ep_ref, snd_ref, rcv_ref, ecnt_ref, out_ref,
              bnd_v, snda_v, rcva_v, snd_v0, snd_v1, rows_v0, rows_v1,
              ep_v0, ep_v1, agg_v, semg0, semg1, seme0, seme1):
    cid = lax.axis_index("c")
    sid = lax.axis_index("s")
    band = cid * 16 + sid

    pltpu.sync_copy(ecnt_ref.at[band], bnd_v)
    ecnt = bnd_v[pl.ds(0, 16)][0]
    estart = band * SLAB
    nch = (ecnt + CHUNK - 1) // CHUNK

    pltpu.sync_copy(snd_ref.at[pl.ds(estart, SLAB)], snda_v)
    pltpu.sync_copy(rcv_ref.at[pl.ds(estart, SLAB)], rcva_v.at[pl.ds(0, SLAB)])

    zero16f = jnp.zeros((16,), jnp.float32)

    def zrow(r, _):
        for c in range(8):
            agg_v[r, pl.ds(16 * c, 16)] = zero16f
        return 0

    lax.fori_loop(0, BAND + 1, zrow, 0)

    iota16 = lax.iota(jnp.int32, 16)
    base = band * BAND
    slots = ((snd_v0, rows_v0, ep_v0, semg0, seme0),
             (snd_v1, rows_v1, ep_v1, semg1, seme1))

    def load(g, slot):
        snd_v, rows_v, ep_v, semg, seme = slots[slot]
        loc = g * CHUNK
        for k in range(8):
            snd_v[pl.ds(16 * k, 16)] = snda_v[pl.ds(loc + 16 * k, 16)]
        pltpu.async_copy(h_ref.at[snd_v], rows_v, semg)
        pltpu.async_copy(ep_ref.at[pl.ds(estart + loc, CHUNK)], ep_v, seme)

    def compute(g, slot):
        snd_v, rows_v, ep_v, semg, seme = slots[slot]
        pltpu.make_async_copy(h_ref.at[snd_v], rows_v, semg).wait()
        pltpu.make_async_copy(ep_ref.at[pl.ds(0, CHUNK)], ep_v, seme).wait()
        loc = g * CHUNK

        @plsc.parallel_loop(0, CHUNK // 16)
        def edge16(q):
            v = rcva_v[pl.ds(loc + q * 16, 16)] - base
            rowsel = jnp.where((v >= 0) & (v < BAND), v, BAND)
            for u in range(16):
                e = q * 16 + u
                rowv = jnp.full((16,), rowsel[u], jnp.int32)
                for c in range(4):
                    bits = ep_v[e, pl.ds(16 * c, 16)]
                    lo = plsc.bitcast(bits << 16, jnp.float32)
                    hi = plsc.bitcast(bits & (-65536), jnp.float32)
                    sl0 = pl.ds(32 * c, 16)
                    sl1 = pl.ds(32 * c + 16, 16)
                    plsc.addupdate_scatter(agg_v, [rowv, iota16 + (32 * c)],
                                           rows_v[e, sl0] * lo)
                    plsc.addupdate_scatter(agg_v, [rowv, iota16 + (32 * c + 16)],
                                           rows_v[e, sl1] * hi)

    @pl.when(nch > 0)
    def _():
        load(0, 0)

    def chunk(g, _):
        slot = lax.rem(g, 2)

        @pl.when(g + 1 < nch)
        def _():
            ns = lax.rem(g + 1, 2)
            lax.cond(ns == 0, lambda: load(g + 1, 0), lambda: load(g + 1, 1))

        lax.cond(slot == 0, lambda: compute(g, 0), lambda: compute(g, 1))
        return 0

    lax.fori_loop(0, nch, chunk, 0)
    pltpu.sync_copy(agg_v.at[pl.ds(0, BAND)], out_ref.at[pl.ds(band * BAND, BAND)])


def _sc_messages(h, eproj, snd, rcv, ecnt):
    mesh = plsc.VectorSubcoreMesh(core_axis_name="c", subcore_axis_name="s")
    f = pl.kernel(
        _msg_body,
        mesh=mesh,
        compiler_params=pltpu.CompilerParams(needs_layout_passes=False),
        out_type=jax.ShapeDtypeStruct((N, D), jnp.float32),
        scratch_types=[
            pltpu.VMEM((16,), jnp.int32),
            pltpu.VMEM((SLAB,), jnp.int32),
            pltpu.VMEM((SLAB + 16,), jnp.int32),
            pltpu.VMEM((CHUNK,), jnp.int32),
            pltpu.VMEM((CHUNK,), jnp.int32),
            pltpu.VMEM((CHUNK, D), jnp.float32),
            pltpu.VMEM((CHUNK, D), jnp.float32),
            pltpu.VMEM((CHUNK, D // 2), jnp.int32),
            pltpu.VMEM((CHUNK, D // 2), jnp.int32),
            pltpu.VMEM((BAND + 1, D), jnp.float32),
            pltpu.SemaphoreType.DMA,
            pltpu.SemaphoreType.DMA,
            pltpu.SemaphoreType.DMA,
            pltpu.SemaphoreType.DMA,
        ],
    )
    return f(h, eproj, snd, rcv, ecnt)


def _update_body(h_ref, p0_ref, wmsg_ref, wuh_ref, wua_ref, o_ref):
    h = h_ref[...]
    agg = p0_ref[...]
    pre = jnp.dot(agg, wmsg_ref[...], preferred_element_type=jnp.float32)
    z = (jnp.dot(h, wuh_ref[...], preferred_element_type=jnp.float32)
         + jnp.dot(pre, wua_ref[...], preferred_element_type=jnp.float32))
    o_ref[...] = h + jnp.maximum(z, 0.0)


def _node_update(h, p0, W_msg, W_u):
    rows = 512
    return pl.pallas_call(
        _update_body,
        grid=(N // rows,),
        in_specs=[
            pl.BlockSpec((rows, D), lambda i: (i, 0)),
            pl.BlockSpec((rows, D), lambda i: (i, 0)),
            pl.BlockSpec((D, D), lambda i: (0, 0)),
            pl.BlockSpec((D, D), lambda i: (0, 0)),
            pl.BlockSpec((D, D), lambda i: (0, 0)),
        ],
        out_specs=pl.BlockSpec((rows, D), lambda i: (i, 0)),
        out_shape=jax.ShapeDtypeStruct((N, D), jnp.float32),
    )(h, p0, W_msg, W_u[:D], W_u[D:])


def kernel(r, node_embed, W_e_0, W_msg_0, W_u_0, W_e_1, W_msg_1, W_u_1,
           W_e_2, W_msg_2, W_u_2):
    rt = r.T
    p = jnp.repeat(jnp.eye(N // 16, dtype=jnp.float32), 16, axis=0)
    cnts = _prefilter(r, rt, p)
    snd, rcv, d2, ecnt = _sc_edges(rt[0], rt[1], rt[2], cnts)
    lo_idx = [32 * (k // 16) + (k % 16) for k in range(D // 2)]
    hi_idx = [c + 16 for c in lo_idx]
    perm = jnp.array(lo_idx + hi_idx, dtype=jnp.int32)
    ep0, ep1, ep2 = _eproj_all(d2, snd, W_e_0[:, perm], W_e_1[:, perm],
                               W_e_2[:, perm])

    h = node_embed
    for eproj, W_msg, W_u in ((ep0, W_msg_0, W_u_0), (ep1, W_msg_1, W_u_1),
                              (ep2, W_msg_2, W_u_2)):
        agg = _sc_messages(h, eproj, snd, rcv, ecnt)
        h = _node_update(h, agg, W_msg, W_u)
    return h


# speedup vs baseline: 1.5581x; 1.0006x over previous
import jax
import jax.numpy as jnp
from jax import lax
from jax.experimental import pallas as pl
from jax.experimental.pallas import tpu as pltpu
from jax.experimental.pallas import tpu_sc as plsc

N = 4096
D = 128
CUT = 0.125
NB = 16
OCC = 163840

N_TILES = 32
CHUNK = 128
BAND = N // N_TILES
SLAB = 6272
ECAP = N_TILES * SLAB
NWORK = 6144
C2EPS = CUT * CUT + 1e-4


def _prefilter_body(rblk_ref, rt_ref, p_ref, o_ref):
    rblk = rblk_ref[...]
    rt = rt_ref[...]
    r2row = jnp.sum(rblk * rblk, axis=1, keepdims=True)
    r2col = jnp.sum(rt * rt, axis=0, keepdims=True)
    dot = jnp.dot(rblk, rt, preferred_element_type=jnp.float32)
    dist2 = (r2row + r2col) - 2.0 * dot
    m = (dist2 < C2EPS).astype(jnp.float32)
    cnt = jnp.dot(m, p_ref[...], preferred_element_type=jnp.float32)
    o_ref[...] = cnt.astype(jnp.int32)


def _prefilter(r, rt, p):
    rows = 256
    return pl.pallas_call(
        _prefilter_body,
        grid=(N // rows,),
        in_specs=[
            pl.BlockSpec((rows, 3), lambda i: (i, 0)),
            pl.BlockSpec((3, N), lambda i: (0, 0)),
            pl.BlockSpec((N, N // 16), lambda i: (0, 0)),
        ],
        out_specs=pl.BlockSpec((rows, N // 16), lambda i: (i, 0)),
        out_shape=jax.ShapeDtypeStruct((N, N // 16), jnp.int32),
    )(r, rt, p)


def _edges_body(xs_ref, ys_ref, zs_ref, cnts_ref, snd_ref, rcv_ref, d2_ref,
                ecnt_ref, x_v, y_v, z_v, r2_v, xb_v, yb_v, zb_v, cl_v, wl_v,
                sndl_v, rcvl_v, d2l_v, ew_v, sem):
    cid = lax.axis_index("c")
    sid = lax.axis_index("s")
    tile = cid * 16 + sid
    r0 = tile * BAND

    pltpu.sync_copy(xs_ref, x_v.at[pl.ds(0, N)])
    pltpu.sync_copy(ys_ref, y_v.at[pl.ds(0, N)])
    pltpu.sync_copy(zs_ref, z_v.at[pl.ds(0, N)])
    pltpu.sync_copy(cnts_ref.at[pl.ds(r0, BAND)], cl_v)

    zeros16 = jnp.zeros((16,), jnp.int32)
    iota16 = lax.iota(jnp.int32, 16)

    def bf16r(v):
        b = plsc.bitcast(v, jnp.int32)
        b = (b + 0x7FFF + ((b >> 16) & 1)) & ~0xFFFF
        return plsc.bitcast(b, jnp.float32)

    def r2row(q, _):
        s = pl.ds(16 * q, 16)
        x = x_v[s]
        y = y_v[s]
        z = z_v[s]
        r2_v[s] = x * x + y * y + z * z
        xb_v[s] = bf16r(x)
        yb_v[s] = bf16r(y)
        zb_v[s] = bf16r(z)
        return 0

    lax.fori_loop(0, N // 16, r2row, 0)

    @plsc.parallel_loop(0, BAND * 16, carry=jnp.int32(0))
    def scan_grp(q, wc):
        i = q // 16
        jj = q % 16
        cvec = cl_v[i, pl.ds(16 * jj, 16)]
        m = cvec > 0
        ids = (i * 256 + 16 * jj) + iota16
        plsc.store_compressed(wl_v.at[pl.ds(wc, 16)], ids, mask=m)
        npos = plsc.all_reduce_population_count(m)
        return wc + npos[0]

    nwork = scan_grp

    @plsc.parallel_loop(0, nwork, carry=jnp.int32(0))
    def visit(e, cur):
        wid = wl_v[pl.ds(e, 16)][0]
        i = wid // 256
        jb = (wid % 256) * 16
        gi = r0 + i
        giv = zeros16 + gi
        xi = x_v[pl.ds(gi, 16)][0]
        yi = y_v[pl.ds(gi, 16)][0]
        zi = z_v[pl.ds(gi, 16)][0]
        r2i = r2_v[pl.ds(gi, 16)][0]
        xbi = xb_v[pl.ds(gi, 16)][0]
        ybi = yb_v[pl.ds(gi, 16)][0]
        zbi = zb_v[pl.ds(gi, 16)][0]
        xj = x_v[pl.ds(jb, 16)]
        yj = y_v[pl.ds(jb, 16)]
        zj = z_v[pl.ds(jb, 16)]
        r2j = r2_v[pl.ds(jb, 16)]
        dot = xbi * xb_v[pl.ds(jb, 16)] + ybi * yb_v[pl.ds(jb, 16)] \
            + zbi * zb_v[pl.ds(jb, 16)]
        d2a = (r2i + r2j) - 2.0 * dot
        jv = jb + iota16
        m = (d2a < CUT * CUT) & (jv != giv)
        dx = xi - xj
        dy = yi - yj
        dz = zi - zj
        d2b = dx * dx + dy * dy + dz * dz
        plsc.store_compressed(sndl_v.at[pl.ds(cur, 16)], jv, mask=m)
        plsc.store_compressed(rcvl_v.at[pl.ds(cur, 16)], giv, mask=m)
        plsc.store_compressed(d2l_v.at[pl.ds(cur, 16)], d2b, mask=m)
        npos = plsc.all_reduce_population_count(m)
        return cur + npos[0]

    ecnt = visit

    def padrow(k, _):
        s = pl.ds(ecnt + 16 * k, 16)
        sndl_v[s] = zeros16
        rcvl_v[s] = zeros16 + N
        d2l_v[s] = jnp.zeros((16,), jnp.float32) + 1.0
        return 0

    lax.fori_loop(0, CHUNK // 16, padrow, 0)

    ew_v[...] = zeros16 + ecnt
    pltpu.sync_copy(ew_v, ecnt_ref.at[tile])
    pltpu.sync_copy(sndl_v, snd_ref.at[pl.ds(tile * SLAB, SLAB)])
    pltpu.sync_copy(rcvl_v, rcv_ref.at[pl.ds(tile * SLAB, SLAB)])
    pltpu.sync_copy(d2l_v, d2_ref.at[pl.ds(tile * SLAB, SLAB)])


def _sc_edges(xs, ys, zs, cnts):
    mesh = plsc.VectorSubcoreMesh(core_axis_name="c", subcore_axis_name="s")
    f = pl.kernel(
        _edges_body,
        mesh=mesh,
        compiler_params=pltpu.CompilerParams(needs_layout_passes=False),
        out_type=[
            jax.ShapeDtypeStruct((ECAP,), jnp.int32),
            jax.ShapeDtypeStruct((ECAP,), jnp.int32),
            jax.ShapeDtypeStruct((ECAP,), jnp.float32),
            jax.ShapeDtypeStruct((N_TILES, 16), jnp.int32),
        ],
        scratch_types=[
            pltpu.VMEM((N + 16,), jnp.float32),
            pltpu.VMEM((N + 16,), jnp.float32),
            pltpu.VMEM((N + 16,), jnp.float32),
            pltpu.VMEM((N + 16,), jnp.float32),
            pltpu.VMEM((N + 16,), jnp.float32),
            pltpu.VMEM((N + 16,), jnp.float32),
            pltpu.VMEM((N + 16,), jnp.float32),
            pltpu.VMEM((BAND, 256), jnp.int32),
            pltpu.VMEM((NWORK + 16,), jnp.int32),
            pltpu.VMEM((SLAB,), jnp.int32),
            pltpu.VMEM((SLAB,), jnp.int32),
            pltpu.VMEM((SLAB,), jnp.float32),
            pltpu.VMEM((16,), jnp.int32),
            pltpu.SemaphoreType.DMA,
        ],
    )
    return f(xs, ys, zs, cnts)


def _bf16_hi(x):
    b = lax.bitcast_convert_type(x, jnp.int32)
    return (b + 0x7FFF + ((b >> 16) & 1)) & (-65536)


def _eproj_body(d2_ref, snd_ref, we0_ref, we1_ref, we2_ref, o0_ref, o1_ref, o2_ref):
    d = jnp.sqrt(d2_ref[...] + 1e-12)
    valid = (snd_ref[...] < N).astype(jnp.float32)
    mu = (CUT / (NB - 1)) * lax.broadcasted_iota(jnp.int32, (1, NB), 1).astype(jnp.float32)
    sigma = CUT / NB
    e16 = jnp.exp(-((d - mu) ** 2) / (2.0 * sigma * sigma)) * valid
    for we_ref, o_ref in ((we0_ref, o0_ref), (we1_ref, o1_ref), (we2_ref, o2_ref)):
        ep = jnp.dot(e16, we_ref[...], preferred_element_type=jnp.float32)
        lo = _bf16_hi(ep[:, :D // 2])
        hi = _bf16_hi(ep[:, D // 2:])
        o_ref[...] = hi | lax.shift_right_logical(lo, 16)


def _eproj_all(d2, snd, W_e_0, W_e_1, W_e_2):
    rows = 2048
    grid = ECAP // rows
    d2 = d2.reshape(ECAP, 1)
    s2 = snd.reshape(ECAP, 1)
    out = jax.ShapeDtypeStruct((ECAP, D // 2), jnp.int32)
    return pl.pallas_call(
        _eproj_body,
        grid=(grid,),
        in_specs=[
            pl.BlockSpec((rows, 1), lambda i: (i, 0)),
            pl.BlockSpec((rows, 1), lambda i: (i, 0)),
            pl.BlockSpec((NB, D), lambda i: (0, 0)),
            pl.BlockSpec((NB, D), lambda i: (0, 0)),
            pl.BlockSpec((NB, D), lambda i: (0, 0)),
        ],
        out_specs=[pl.BlockSpec((rows, D // 2), lambda i: (i, 0))] * 3,
        out_shape=[out, out, out],
    )(d2, s2, W_e_0, W_e_1, W_e_2)


BAND = N // N_TILES


def _msg_body(h_ref, ep_ref, snd_ref, rcv_ref, ecnt_ref, out_ref,
              bnd_v, snda_v, rcva_v, snd_v0, snd_v1, rows_v0, rows_v1,
              ep_v0, ep_v1, agg_v, semg0, semg1, seme0, seme1):
    cid = lax.axis_index("c")
    sid = lax.axis_index("s")
    band = cid * 16 + sid

    pltpu.sync_copy(ecnt_ref.at[band], bnd_v)
    ecnt = bnd_v[pl.ds(0, 16)][0]
    estart = band * SLAB
    nch = (ecnt + CHUNK - 1) // CHUNK

    pltpu.sync_copy(snd_ref.at[pl.ds(estart, SLAB)], snda_v)
    pltpu.sync_copy(rcv_ref.at[pl.ds(estart, SLAB)], rcva_v.at[pl.ds(0, SLAB)])

    zero16f = jnp.zeros((16,), jnp.float32)

    def zrow(r, _):
        for c in range(8):
            agg_v[r, pl.ds(16 * c, 16)] = zero16f
        return 0

    lax.fori_loop(0, BAND + 1, zrow, 0)

    iota16 = lax.iota(jnp.int32, 16)
    base = band * BAND
    slots = ((snd_v0, rows_v0, ep_v0, semg0, seme0),
             (snd_v1, rows_v1, ep_v1, semg1, seme1))

    def load(g, slot):
        snd_v, rows_v, ep_v, semg, seme = slots[slot]
        loc = g * CHUNK
        for k in range(8):
            snd_v[pl.ds(16 * k, 16)] = snda_v[pl.ds(loc + 16 * k, 16)]
        pltpu.async_copy(h_ref.at[snd_v], rows_v, semg)
        pltpu.async_copy(ep_ref.at[pl.ds(estart + loc, CHUNK)], ep_v, seme)

    def compute(g, slot):
        snd_v, rows_v, ep_v, semg, seme = slots[slot]
        pltpu.make_async_copy(h_ref.at[snd_v], rows_v, semg).wait()
        pltpu.make_async_copy(ep_ref.at[pl.ds(0, CHUNK)], ep_v, seme).wait()
        loc = g * CHUNK

        @plsc.parallel_loop(0, CHUNK // 16)
        def edge16(q):
            v = rcva_v[pl.ds(loc + q * 16, 16)] - base
            rowsel = jnp.where((v >= 0) & (v < BAND), v, BAND)
            for u in range(16):
                e = q * 16 + u
                rowv = jnp.full((16,), rowsel[u], jnp.int32)
                for c in range(4):
                    bits = ep_v[e, pl.ds(16 * c, 16)]
                    lo = plsc.bitcast(bits << 16, jnp.float32)
                    hi = plsc.bitcast(bits & (-65536), jnp.float32)
                    sl0 = pl.ds(32 * c, 16)
                    sl1 = pl.ds(32 * c + 16, 16)
                    plsc.addupdate_scatter(agg_v, [rowv, iota16 + (32 * c)],
                                           rows_v[e, sl0] * lo)
                    plsc.addupdate_scatter(agg_v, [rowv, iota16 + (32 * c + 16)],
                                           rows_v[e, sl1] * hi)

    @pl.when(nch > 0)
    def _():
        load(0, 0)

    def chunk(g, _):
        slot = lax.rem(g, 2)

        @pl.when(g + 1 < nch)
        def _():
            ns = lax.rem(g + 1, 2)
            lax.cond(ns == 0, lambda: load(g + 1, 0), lambda: load(g + 1, 1))

        lax.cond(slot == 0, lambda: compute(g, 0), lambda: compute(g, 1))
        return 0

    lax.fori_loop(0, nch, chunk, 0)
    pltpu.sync_copy(agg_v.at[pl.ds(0, BAND)], out_ref.at[pl.ds(band * BAND, BAND)])


def _sc_messages(h, eproj, snd, rcv, ecnt):
    mesh = plsc.VectorSubcoreMesh(core_axis_name="c", subcore_axis_name="s")
    f = pl.kernel(
        _msg_body,
        mesh=mesh,
        compiler_params=pltpu.CompilerParams(needs_layout_passes=False),
        out_type=jax.ShapeDtypeStruct((N, D), jnp.float32),
        scratch_types=[
            pltpu.VMEM((16,), jnp.int32),
            pltpu.VMEM((SLAB,), jnp.int32),
            pltpu.VMEM((SLAB + 16,), jnp.int32),
            pltpu.VMEM((CHUNK,), jnp.int32),
            pltpu.VMEM((CHUNK,), jnp.int32),
            pltpu.VMEM((CHUNK, D), jnp.float32),
            pltpu.VMEM((CHUNK, D), jnp.float32),
            pltpu.VMEM((CHUNK, D // 2), jnp.int32),
            pltpu.VMEM((CHUNK, D // 2), jnp.int32),
            pltpu.VMEM((BAND + 1, D), jnp.float32),
            pltpu.SemaphoreType.DMA,
            pltpu.SemaphoreType.DMA,
            pltpu.SemaphoreType.DMA,
            pltpu.SemaphoreType.DMA,
        ],
    )
    return f(h, eproj, snd, rcv, ecnt)


def _update_body(h_ref, p0_ref, wmsg_ref, wuh_ref, wua_ref, o_ref):
    h = h_ref[...]
    agg = p0_ref[...]
    pre = jnp.dot(agg, wmsg_ref[...], preferred_element_type=jnp.float32)
    z = (jnp.dot(h, wuh_ref[...], preferred_element_type=jnp.float32)
         + jnp.dot(pre, wua_ref[...], preferred_element_type=jnp.float32))
    o_ref[...] = h + jnp.maximum(z, 0.0)


def _node_update(h, p0, W_msg, W_u):
    rows = 512
    return pl.pallas_call(
        _update_body,
        grid=(N // rows,),
        in_specs=[
            pl.BlockSpec((rows, D), lambda i: (i, 0)),
            pl.BlockSpec((rows, D), lambda i: (i, 0)),
            pl.BlockSpec((D, D), lambda i: (0, 0)),
            pl.BlockSpec((D, D), lambda i: (0, 0)),
            pl.BlockSpec((D, D), lambda i: (0, 0)),
        ],
        out_specs=pl.BlockSpec((rows, D), lambda i: (i, 0)),
        out_shape=jax.ShapeDtypeStruct((N, D), jnp.float32),
    )(h, p0, W_msg, W_u[:D], W_u[D:])


def kernel(r, node_embed, W_e_0, W_msg_0, W_u_0, W_e_1, W_msg_1, W_u_1,
           W_e_2, W_msg_2, W_u_2):
    rt = r.T
    p = jnp.repeat(jnp.eye(N // 16, dtype=jnp.float32), 16, axis=0)
    cnts = _prefilter(r, rt, p)
    snd, rcv, d2, ecnt = _sc_edges(rt[0], rt[1], rt[2], cnts)
    lo_idx = [32 * (k // 16) + (k % 16) for k in range(D // 2)]
    hi_idx = [c + 16 for c in lo_idx]
    perm = jnp.array(lo_idx + hi_idx, dtype=jnp.int32)
    ep0, ep1, ep2 = _eproj_all(d2, snd, W_e_0[:, perm], W_e_1[:, perm],
                               W_e_2[:, perm])

    h = node_embed
    for eproj, W_msg, W_u in ((ep0, W_msg_0, W_u_0), (ep1, W_msg_1, W_u_1),
                              (ep2, W_msg_2, W_u_2)):
        agg = _sc_messages(h, eproj, snd, rcv, ecnt)
        h = _node_update(h, agg, W_msg, W_u)
    return h
